# baseline (speedup 1.0000x reference)
import functools

import jax
import jax.numpy as jnp
from jax import lax
from jax.experimental import pallas as pl
from jax.experimental.pallas import tpu as pltpu
from jax.experimental.pallas import tpu_sc as plsc

_VOCAB = 1000000
_DIM = 64
_BATCH = 4096
_HIST = 200

_B = _BATCH * _HIST
_NC, _NS = 2, 16
_NW = _NC * _NS
_B_PER_W = _B // _NW
_CHUNK = 128
_N_CHUNKS = _B_PER_W // _CHUNK

_mesh = plsc.VectorSubcoreMesh(
    core_axis_name="c", subcore_axis_name="s",
    num_cores=_NC, num_subcores=_NS,
)


@functools.partial(
    pl.kernel,
    out_type=jax.ShapeDtypeStruct((_B, _DIM), jnp.float32),
    mesh=_mesh,
    scratch_types=[
        pltpu.VMEM((_CHUNK,), jnp.int32),
        pltpu.VMEM((_CHUNK, _DIM), jnp.float32),
        pltpu.SemaphoreType.DMA,
    ],
    compiler_params=pltpu.CompilerParams(use_tc_tiling_on_sc=False),
)
def _gather_kernel(idx_hbm, table_hbm, out_hbm, idx_v, rows_v, sem):
    wid = lax.axis_index("s") * _NC + lax.axis_index("c")
    base = wid * _B_PER_W

    @pl.loop(0, _N_CHUNKS)
    def _chunk(g):
        off = base + g * _CHUNK
        pltpu.sync_copy(idx_hbm.at[pl.ds(off, _CHUNK)], idx_v)
        pltpu.async_copy(table_hbm.at[idx_v], rows_v, sem).wait()
        pltpu.sync_copy(rows_v, out_hbm.at[pl.ds(off, _CHUNK)])


def kernel(inputs, table):
    idx = inputs.reshape(_B)
    out = _gather_kernel(idx, table)
    return out.reshape(_BATCH, _HIST, _DIM)


# speedup vs baseline: 1.1609x; 1.1609x over previous
import functools

import jax
import jax.numpy as jnp
from jax import lax
from jax.experimental import pallas as pl
from jax.experimental.pallas import tpu as pltpu
from jax.experimental.pallas import tpu_sc as plsc

_VOCAB = 1000000
_DIM = 64
_BATCH = 4096
_HIST = 200

_B = _BATCH * _HIST
_NC, _NS = 2, 16
_NW = _NC * _NS
_B_PER_W = _B // _NW
_CHUNK = 128
_N_CHUNKS = _B_PER_W // _CHUNK
_NBUF = 4

_mesh = plsc.VectorSubcoreMesh(
    core_axis_name="c", subcore_axis_name="s",
    num_cores=_NC, num_subcores=_NS,
)


@functools.partial(
    pl.kernel,
    out_type=jax.ShapeDtypeStruct((_B, _DIM), jnp.float32),
    mesh=_mesh,
    scratch_types=[
        pltpu.VMEM((_NBUF, _CHUNK), jnp.int32),
        [pltpu.VMEM((_CHUNK, _DIM), jnp.float32) for _ in range(_NBUF)],
        [pltpu.SemaphoreType.DMA for _ in range(_NBUF)],
        [pltpu.SemaphoreType.DMA for _ in range(_NBUF)],
    ],
    compiler_params=pltpu.CompilerParams(use_tc_tiling_on_sc=False),
)
def _gather_kernel(idx_hbm, table_hbm, out_hbm, idx_v, rows, gsem, wsem):
    wid = lax.axis_index("s") * _NC + lax.axis_index("c")
    chunk0 = wid * _N_CHUNKS
    base = wid * _B_PER_W

    pltpu.sync_copy(idx_hbm.at[pl.ds(chunk0, _NBUF)], idx_v)
    for b in range(_NBUF):
        pltpu.async_copy(table_hbm.at[idx_v.at[b]], rows[b], gsem[b])

    @pl.loop(0, _N_CHUNKS, step=_NBUF)
    def _slab(g0):
        for b in range(_NBUF):
            pltpu.make_async_copy(
                table_hbm.at[idx_v.at[b]], rows[b], gsem[b]).wait()
            pltpu.async_copy(
                rows[b], out_hbm.at[pl.ds(base + (g0 + b) * _CHUNK, _CHUNK)],
                wsem[b])

        @pl.when(g0 + _NBUF < _N_CHUNKS)
        def _next():
            pltpu.sync_copy(
                idx_hbm.at[pl.ds(chunk0 + g0 + _NBUF, _NBUF)], idx_v)
            for b in range(_NBUF):
                pltpu.make_async_copy(
                    rows[b],
                    out_hbm.at[pl.ds(base + (g0 + b) * _CHUNK, _CHUNK)],
                    wsem[b]).wait()
                pltpu.async_copy(table_hbm.at[idx_v.at[b]], rows[b], gsem[b])

    last0 = _N_CHUNKS - _NBUF
    for b in range(_NBUF):
        pltpu.make_async_copy(
            rows[b],
            out_hbm.at[pl.ds(base + (last0 + b) * _CHUNK, _CHUNK)],
            wsem[b]).wait()


def kernel(inputs, table):
    idx = inputs.reshape(_B // _CHUNK, _CHUNK)
    out = _gather_kernel(idx, table)
    return out.reshape(_BATCH, _HIST, _DIM)


# speedup vs baseline: 1.1768x; 1.0136x over previous
import functools

import jax
import jax.numpy as jnp
from jax import lax
from jax.experimental import pallas as pl
from jax.experimental.pallas import tpu as pltpu
from jax.experimental.pallas import tpu_sc as plsc

_VOCAB = 1000000
_DIM = 64
_BATCH = 4096
_HIST = 200

_B = _BATCH * _HIST
_NC, _NS = 2, 16
_NW = _NC * _NS
_B_PER_W = _B // _NW
_CHUNK = 256
_N_CHUNKS = _B_PER_W // _CHUNK
_NBUF = 4

_mesh = plsc.VectorSubcoreMesh(
    core_axis_name="c", subcore_axis_name="s",
    num_cores=_NC, num_subcores=_NS,
)


@functools.partial(
    pl.kernel,
    out_type=jax.ShapeDtypeStruct((_B, _DIM), jnp.float32),
    mesh=_mesh,
    scratch_types=[
        pltpu.VMEM((_NBUF, _CHUNK), jnp.int32),
        [pltpu.VMEM((_CHUNK, _DIM), jnp.float32) for _ in range(_NBUF)],
        [pltpu.SemaphoreType.DMA for _ in range(_NBUF)],
        [pltpu.SemaphoreType.DMA for _ in range(_NBUF)],
    ],
    compiler_params=pltpu.CompilerParams(use_tc_tiling_on_sc=False),
)
def _gather_kernel(idx_hbm, table_hbm, out_hbm, idx_v, rows, gsem, wsem):
    wid = lax.axis_index("s") * _NC + lax.axis_index("c")
    chunk0 = wid * _N_CHUNKS
    base = wid * _B_PER_W

    pltpu.sync_copy(idx_hbm.at[pl.ds(chunk0, _NBUF)], idx_v)
    for b in range(_NBUF):
        pltpu.async_copy(table_hbm.at[idx_v.at[b]], rows[b], gsem[b])

    @pl.loop(0, _N_CHUNKS, step=_NBUF)
    def _slab(g0):
        for b in range(_NBUF):
            pltpu.make_async_copy(
                table_hbm.at[idx_v.at[b]], rows[b], gsem[b]).wait()
            pltpu.async_copy(
                rows[b], out_hbm.at[pl.ds(base + (g0 + b) * _CHUNK, _CHUNK)],
                wsem[b])

        @pl.when(g0 + _NBUF < _N_CHUNKS)
        def _next():
            pltpu.sync_copy(
                idx_hbm.at[pl.ds(chunk0 + g0 + _NBUF, _NBUF)], idx_v)
            for b in range(_NBUF):
                pltpu.make_async_copy(
                    rows[b],
                    out_hbm.at[pl.ds(base + (g0 + b) * _CHUNK, _CHUNK)],
                    wsem[b]).wait()
                pltpu.async_copy(table_hbm.at[idx_v.at[b]], rows[b], gsem[b])

    last0 = _N_CHUNKS - _NBUF
    for b in range(_NBUF):
        pltpu.make_async_copy(
            rows[b],
            out_hbm.at[pl.ds(base + (last0 + b) * _CHUNK, _CHUNK)],
            wsem[b]).wait()


def kernel(inputs, table):
    idx = inputs.reshape(_B // _CHUNK, _CHUNK)
    out = _gather_kernel(idx, table)
    return out.reshape(_BATCH, _HIST, _DIM)
